# baseline (speedup 1.0000x reference)
import jax
import jax.numpy as jnp
from jax.experimental import pallas as pl
from jax.experimental.pallas import tpu as pltpu

_BB = 512


def _shift_sel():
    j = jnp.arange(3)[:, None, None]
    q = jnp.arange(8)[None, :, None]
    n = jnp.arange(8)[None, None, :]
    return ((q == n + j - 1) & (q < 7) & (n < 7)).astype(jnp.float32)


def _prep_weights(w2pad, wc1, wc2):
    bf = jnp.bfloat16
    sel = _shift_sel()

    w2r = w2pad.reshape(256, 32, 128)[:, :, :49].reshape(256, 32, 7, 7)
    w2t = jnp.transpose(w2r, (0, 2, 3, 1)).reshape(256, 7, 224)
    w2t = jnp.pad(w2t, ((0, 0), (1, 1), (0, 32)))
    w2b = w2t.reshape(256, 2304).astype(bf)

    g1 = wc1.reshape(256, 9, 32).transpose(1, 2, 0).reshape(3, 3, 32, 256)
    wr1 = jnp.einsum('ajcd,jqn->aqcnd', g1, sel[:, :, :7])
    wr1 = wr1.reshape(768, 1792).astype(bf)

    g2 = (wc2.reshape(4, 4, 9, 256).transpose(2, 3, 0, 1)
          .reshape(3, 3, 256, 4, 4))
    selr = (jnp.arange(4)[None, :, None] ==
            jnp.arange(3)[:, None, None] +
            jnp.arange(2)[None, None, :]).astype(jnp.float32)
    wr2p = jnp.einsum('ajeyx,arm,jqn->rqemynx', g2, selr, sel[:, :7, :7])
    wr2p = wr2p.reshape(7168, 224).astype(bf)
    wr2l = jnp.einsum('ajeyx,jqn->aqeynx', g2, sel[:, :7, :7])
    wr2l = wr2l.reshape(5376, 112).astype(bf)
    return w2b, wr1, wr2p, wr2l


def _body(zp_ref, zs_ref, t_ref, w1_ref, w2_ref, wr1_ref, wr2p_ref, wr2l_ref,
          img_ref, sse_ref):
    f32 = jnp.float32
    bf = jnp.bfloat16

    z = jnp.concatenate([zp_ref[...], zs_ref[...]], axis=1).astype(bf)
    h1 = jnp.dot(z, w1_ref[...], preferred_element_type=f32)
    h1 = jnp.maximum(h1.astype(bf), 0)
    h2 = jnp.dot(h1, w2_ref[...], preferred_element_type=f32)
    h2 = jnp.maximum(h2.astype(bf), 0)

    wr1 = wr1_ref[...]
    rows = []
    for m in range(7):
        o = jnp.dot(h2[:, m * 256:(m + 3) * 256], wr1,
                    preferred_element_type=f32)
        rows.append(jnp.maximum(o.astype(bf), 0))

    zblk = jnp.zeros_like(rows[0])
    o1pad = jnp.concatenate([zblk] + rows + [zblk],
                            axis=1)
    wr2p = wr2p_ref[...]
    outs = []
    for t in range(3):
        lhs = o1pad[:, t * 3584:t * 3584 + 7168]
        outs.append(jnp.dot(lhs, wr2p,
                            preferred_element_type=f32))
    lhs_l = o1pad[:, 6 * 1792:9 * 1792]
    outs.append(jnp.dot(lhs_l, wr2l_ref[...],
                        preferred_element_type=f32))
    img = jnp.concatenate(outs, axis=1)
    img_ref[...] = img.reshape(img.shape[0], 1, 28, 28)

    d = t_ref[...].astype(f32) - img
    sse_ref[...] = jnp.sum(d * d, axis=1, keepdims=True)


def kernel(w1, w2pad, wc1, wc2, images, z_private, z_shared):
    f32 = jnp.float32
    bf = jnp.bfloat16
    B = images.shape[0]
    bb = _BB if B % _BB == 0 else B

    zp_dim, zs_dim = z_private.shape[1], z_shared.shape[1]
    w1b = w1.astype(bf)
    w2b, wr1, wr2p, wr2l = _prep_weights(w2pad, wc1, wc2)

    targ = images.astype(bf).reshape(B, 784)

    nsteps = B // bb
    ncores = 2 if nsteps % 2 == 0 else 1
    inner = nsteps // ncores

    def blk(i, j):
        return (i * inner + j, 0)

    def rep(i, j):
        return (0, 0)

    img, sse = pl.pallas_call(
        _body,
        out_shape=(jax.ShapeDtypeStruct((B, 1, 28, 28), f32),
                   jax.ShapeDtypeStruct((B, 1), f32)),
        grid=(ncores, inner),
        in_specs=[
            pl.BlockSpec((bb, zp_dim), blk),
            pl.BlockSpec((bb, zs_dim), blk),
            pl.BlockSpec((bb, 784), blk),
            pl.BlockSpec((zp_dim + zs_dim, 256), rep),
            pl.BlockSpec((256, 2304), rep),
            pl.BlockSpec((768, 1792), rep),
            pl.BlockSpec((7168, 224), rep),
            pl.BlockSpec((5376, 112), rep),
        ],
        out_specs=(
            pl.BlockSpec((bb, 1, 28, 28), lambda i, j: (i * inner + j, 0, 0, 0)),
            pl.BlockSpec((bb, 1), blk),
        ),
        compiler_params=pltpu.CompilerParams(
            dimension_semantics=("parallel", "arbitrary"),
            vmem_limit_bytes=58 * 1024 * 1024),
    )(z_private, z_shared, targ, w1b, w2b, wr1, wr2p, wr2l)

    return img, sse.reshape(B)


# speedup vs baseline: 1.0690x; 1.0690x over previous
import jax
import jax.numpy as jnp
from jax.experimental import pallas as pl
from jax.experimental.pallas import tpu as pltpu

_BB = 512


def _shift_sel():
    j = jnp.arange(3)[:, None, None]
    q = jnp.arange(8)[None, :, None]
    n = jnp.arange(8)[None, None, :]
    return ((q == n + j - 1) & (q < 7) & (n < 7)).astype(jnp.float32)


def _prep_weights(w2pad, wc1, wc2):
    bf = jnp.bfloat16
    sel = _shift_sel()

    w2r = w2pad.reshape(256, 32, 128)[:, :, :49].reshape(256, 32, 7, 7)
    w2t = jnp.transpose(w2r, (0, 2, 3, 1)).reshape(256, 7, 224)
    w2t = jnp.pad(w2t, ((0, 0), (1, 1), (0, 32)))
    w2b = w2t.reshape(256, 2304).astype(bf)

    g1 = wc1.reshape(256, 9, 32).transpose(1, 2, 0).reshape(3, 3, 32, 256)
    wr1 = jnp.einsum('ajcd,jqn->aqcnd', g1, sel[:, :, :7])
    wr1 = wr1.reshape(768, 1792).astype(bf)

    g2 = (wc2.reshape(4, 4, 9, 256).transpose(2, 3, 0, 1)
          .reshape(3, 3, 256, 4, 4))
    wr2l = jnp.einsum('ajeyx,jqn->aqeynx', g2, sel[:, :7, :7])
    wr2l = wr2l.reshape(5376, 112).astype(bf)
    wr2p = jnp.concatenate([jnp.pad(wr2l, ((0, 1792), (0, 0))),
                            jnp.pad(wr2l, ((1792, 0), (0, 0)))], axis=1)
    return w2b, wr1, wr2p, wr2l


def _body(zp_ref, zs_ref, t_ref, w1_ref, w2_ref, wr1_ref, wr2p_ref, wr2l_ref,
          img_ref, sse_ref):
    f32 = jnp.float32
    bf = jnp.bfloat16

    z = jnp.concatenate([zp_ref[...], zs_ref[...]], axis=1).astype(bf)
    h1 = jnp.dot(z, w1_ref[...], preferred_element_type=f32)
    h1 = jnp.maximum(h1.astype(bf), 0)
    h2 = jnp.dot(h1, w2_ref[...], preferred_element_type=f32)
    h2 = jnp.maximum(h2.astype(bf), 0)

    wr1 = wr1_ref[...]
    rows = []
    for m in range(7):
        o = jnp.dot(h2[:, m * 256:(m + 3) * 256], wr1,
                    preferred_element_type=f32)
        rows.append(jnp.maximum(o.astype(bf), 0))

    zblk = jnp.zeros_like(rows[0])
    o1pad = jnp.concatenate([zblk] + rows + [zblk],
                            axis=1)
    wr2p = wr2p_ref[...]
    outs = []
    for t in range(3):
        lhs = o1pad[:, t * 3584:t * 3584 + 7168]
        outs.append(jnp.dot(lhs, wr2p,
                            preferred_element_type=f32))
    lhs_l = o1pad[:, 6 * 1792:9 * 1792]
    outs.append(jnp.dot(lhs_l, wr2l_ref[...],
                        preferred_element_type=f32))
    img = jnp.concatenate(outs, axis=1)
    img_ref[...] = img

    d = t_ref[...].astype(f32) - img
    sse_ref[...] = jnp.sum(d * d, axis=1, keepdims=True)


def kernel(w1, w2pad, wc1, wc2, images, z_private, z_shared):
    f32 = jnp.float32
    bf = jnp.bfloat16
    B = images.shape[0]
    bb = _BB if B % _BB == 0 else B

    zp_dim, zs_dim = z_private.shape[1], z_shared.shape[1]
    w1b = w1.astype(bf)
    w2b, wr1, wr2p, wr2l = _prep_weights(w2pad, wc1, wc2)

    targ = images.astype(bf).reshape(B, 784)

    nsteps = B // bb
    ncores = 2 if nsteps % 2 == 0 else 1
    inner = nsteps // ncores

    def blk(i, j):
        return (i * inner + j, 0)

    def rep(i, j):
        return (0, 0)

    img_flat, sse = pl.pallas_call(
        _body,
        out_shape=(jax.ShapeDtypeStruct((B, 784), f32),
                   jax.ShapeDtypeStruct((B, 1), f32)),
        grid=(ncores, inner),
        in_specs=[
            pl.BlockSpec((bb, zp_dim), blk),
            pl.BlockSpec((bb, zs_dim), blk),
            pl.BlockSpec((bb, 784), blk),
            pl.BlockSpec((zp_dim + zs_dim, 256), rep),
            pl.BlockSpec((256, 2304), rep),
            pl.BlockSpec((768, 1792), rep),
            pl.BlockSpec((7168, 224), rep),
            pl.BlockSpec((5376, 112), rep),
        ],
        out_specs=(
            pl.BlockSpec((bb, 784), blk),
            pl.BlockSpec((bb, 1), blk),
        ),
        compiler_params=pltpu.CompilerParams(
            dimension_semantics=("parallel", "arbitrary"),
            vmem_limit_bytes=58 * 1024 * 1024),
    )(z_private, z_shared, targ, w1b, w2b, wr1, wr2p, wr2l)

    return img_flat.reshape(B, 1, 28, 28), sse.reshape(B)


# speedup vs baseline: 1.0724x; 1.0032x over previous
import jax
import jax.numpy as jnp
from jax.experimental import pallas as pl
from jax.experimental.pallas import tpu as pltpu

_BB = 512


def _shift_sel():
    j = jnp.arange(3)[:, None, None]
    q = jnp.arange(8)[None, :, None]
    n = jnp.arange(8)[None, None, :]
    return ((q == n + j - 1) & (q < 7) & (n < 7)).astype(jnp.float32)


def _prep_weights(w2pad, wc1, wc2):
    bf = jnp.bfloat16
    sel = _shift_sel()

    w2r = w2pad.reshape(256, 32, 128)[:, :, :49].reshape(256, 32, 7, 7)
    w2t = jnp.transpose(w2r, (0, 2, 3, 1)).reshape(256, 7, 224)
    w2t = jnp.pad(w2t, ((0, 0), (1, 1), (0, 32)))
    w2b = w2t.reshape(256, 2304).astype(bf)

    g1 = wc1.reshape(256, 9, 32).transpose(1, 2, 0).reshape(3, 3, 32, 256)
    wr1 = jnp.einsum('ajcd,jqn->aqcnd', g1, sel[:, :, :7])
    wr1 = wr1.reshape(768, 1792).astype(bf)

    g2 = (wc2.reshape(4, 4, 9, 256).transpose(2, 3, 0, 1)
          .reshape(3, 3, 256, 4, 4))
    wr2l = jnp.einsum('ajeyx,jqn->aqeynx', g2, sel[:, :7, :7])
    wr2l = wr2l.reshape(5376, 112).astype(bf)
    wr2p = jnp.concatenate([jnp.pad(wr2l, ((0, 1792), (0, 0))),
                            jnp.pad(wr2l, ((1792, 0), (0, 0)))], axis=1)
    return w2b, wr1, wr2p, wr2l


def _body(zp_ref, zs_ref, t_ref, w1_ref, w2_ref, wr1_ref, wr2p_ref, wr2l_ref,
          img_ref, sse_ref):
    f32 = jnp.float32
    bf = jnp.bfloat16

    z = jnp.concatenate([zp_ref[...], zs_ref[...]], axis=1).astype(bf)
    h1 = jnp.dot(z, w1_ref[...], preferred_element_type=f32)
    h1 = jnp.maximum(h1.astype(bf), 0)
    h2 = jnp.dot(h1, w2_ref[...], preferred_element_type=f32)
    h2 = jnp.maximum(h2.astype(bf), 0)

    bbn = h2.shape[0]
    lhs1 = jnp.concatenate([h2[:, m * 256:(m + 3) * 256] for m in range(7)],
                           axis=0)
    o1 = jnp.dot(lhs1, wr1_ref[...], preferred_element_type=f32)
    o1 = jnp.maximum(o1.astype(bf), 0)
    rows = [o1[m * bbn:(m + 1) * bbn] for m in range(7)]

    zblk = jnp.zeros_like(rows[0])
    o1pad = jnp.concatenate([zblk] + rows + [zblk],
                            axis=1)
    wr2p = wr2p_ref[...]
    outs = []
    for t in range(3):
        lhs = o1pad[:, t * 3584:t * 3584 + 7168]
        outs.append(jnp.dot(lhs, wr2p,
                            preferred_element_type=f32))
    lhs_l = o1pad[:, 6 * 1792:9 * 1792]
    outs.append(jnp.dot(lhs_l, wr2l_ref[...],
                        preferred_element_type=f32))
    img = jnp.concatenate(outs, axis=1)
    img_ref[...] = img

    d = t_ref[...].astype(f32) - img
    sse_ref[...] = jnp.sum(d * d, axis=1, keepdims=True)


def kernel(w1, w2pad, wc1, wc2, images, z_private, z_shared):
    f32 = jnp.float32
    bf = jnp.bfloat16
    B = images.shape[0]
    bb = _BB if B % _BB == 0 else B

    zp_dim, zs_dim = z_private.shape[1], z_shared.shape[1]
    w1b = w1.astype(bf)
    w2b, wr1, wr2p, wr2l = _prep_weights(w2pad, wc1, wc2)

    targ = images.astype(bf).reshape(B, 784)

    nsteps = B // bb
    ncores = 2 if nsteps % 2 == 0 else 1
    inner = nsteps // ncores

    def blk(i, j):
        return (i * inner + j, 0)

    def rep(i, j):
        return (0, 0)

    img_flat, sse = pl.pallas_call(
        _body,
        out_shape=(jax.ShapeDtypeStruct((B, 784), f32),
                   jax.ShapeDtypeStruct((B, 1), f32)),
        grid=(ncores, inner),
        in_specs=[
            pl.BlockSpec((bb, zp_dim), blk),
            pl.BlockSpec((bb, zs_dim), blk),
            pl.BlockSpec((bb, 784), blk),
            pl.BlockSpec((zp_dim + zs_dim, 256), rep),
            pl.BlockSpec((256, 2304), rep),
            pl.BlockSpec((768, 1792), rep),
            pl.BlockSpec((7168, 224), rep),
            pl.BlockSpec((5376, 112), rep),
        ],
        out_specs=(
            pl.BlockSpec((bb, 784), blk),
            pl.BlockSpec((bb, 1), blk),
        ),
        compiler_params=pltpu.CompilerParams(
            dimension_semantics=("parallel", "arbitrary"),
            vmem_limit_bytes=58 * 1024 * 1024),
    )(z_private, z_shared, targ, w1b, w2b, wr1, wr2p, wr2l)

    return img_flat.reshape(B, 1, 28, 28), sse.reshape(B)
